# baseline (speedup 1.0000x reference)
import functools

import jax
import jax.numpy as jnp
from jax import lax
from jax.experimental import pallas as pl
from jax.experimental.pallas import tpu as pltpu
from jax.experimental.pallas import tpu_sc as plsc

N = 10000
NP = 10240
E = 320000
NW = 32
EW = E // NW
K = 128
GB = 8
NG = 10
NPAIR = NG // 2
EWP = NG * GB * K
PAD = EWP - EW
STRIPE = NP // 16
ZR = 64
EPS = 1e-5


def _sc_segment_sum(C):
    mesh = plsc.VectorSubcoreMesh(core_axis_name="c", subcore_axis_name="s")

    @functools.partial(
        pl.kernel,
        out_type=jax.ShapeDtypeStruct((2, C, NP, 128), jnp.float32),
        mesh=mesh,
        scratch_types=[
            pltpu.VMEM((ZR, 128), jnp.float32),
            pltpu.VMEM((2, GB, K), jnp.int32),
            pltpu.VMEM((2, GB, K), jnp.int32),
            pltpu.VMEM((K, 128), jnp.float32),
            pltpu.VMEM((K, 128), jnp.float32),
            pltpu.VMEM_SHARED((NP, 128), jnp.float32),
            pltpu.SemaphoreType.DMA,
            pltpu.SemaphoreType.DMA,
            pltpu.SemaphoreType.DMA,
            pltpu.SemaphoreType.DMA,
            pltpu.SemaphoreType.DMA,
            pltpu.SemaphoreType.DMA,
            pltpu.SemaphoreType.DMA,
            pltpu.SemaphoreType.DMA,
        ],
    )
    def k(h4_hbm, gs_hbm, out_hbm, zbuf, gsl0, gsl1, rowsA, rowsB, acc,
          si0, si1, srA, srB, sz, sw, ssA, ssB):
        cid = lax.axis_index("c")
        sid = lax.axis_index("s")
        wid = sid * 2 + cid
        row0 = sid * STRIPE
        gsl = (gsl0, gsl1)
        sem_i = (si0, si1)
        rows = (rowsA, rowsB)
        sem_r = (srA, srB)
        sem_s = (ssA, ssB)

        def zrow(r, carry):
            for j in range(8):
                zbuf[r, pl.ds(j * 16, 16)] = jnp.zeros((16,), jnp.float32)
            return carry

        lax.fori_loop(0, ZR, zrow, 0)

        def zero_issue():
            for z in range(STRIPE // ZR):
                pltpu.async_copy(zbuf, acc.at[pl.ds(row0 + z * ZR, ZR)], sz)

        def zero_wait():
            for z in range(STRIPE // ZR):
                pltpu.make_async_copy(
                    zbuf, acc.at[pl.ds(row0 + z * ZR, ZR)], sz).wait()

        def prologue(c):
            pltpu.sync_copy(gs_hbm.at[c, wid, 0], gsl0)
            pltpu.async_copy(gs_hbm.at[c, wid, 1], gsl1, si1)
            pltpu.async_copy(h4_hbm.at[gsl0.at[0, 0]], rowsA, srA)

        def wb_slices(c):
            return acc.at[pl.ds(row0, STRIPE)], out_hbm.at[
                cid, c, pl.ds(row0, STRIPE)]

        for c in range(C):
            if c == 0:
                zero_issue()
                prologue(0)
                zero_wait()
                plsc.subcore_barrier()

            def pair(i, carry):
                for gpar in (0, 1):
                    myg = gsl[gpar]
                    for j in range(GB):
                        p = j % 2

                        def wait_prev_scatter():
                            pltpu.make_async_copy(
                                rows[p ^ 1], acc.at[myg.at[1, j]],
                                sem_s[p ^ 1]).wait()

                        if j < GB - 1:
                            if j == 0 and gpar == 0:
                                @pl.when(i > 0)
                                def _():
                                    wait_prev_scatter()
                            else:
                                wait_prev_scatter()
                            pltpu.async_copy(h4_hbm.at[myg.at[0, j + 1]],
                                             rows[p ^ 1], sem_r[p ^ 1])
                        elif gpar == 0:
                            pltpu.make_async_copy(
                                gs_hbm.at[c, wid, 2 * i + 1], gsl1, si1
                            ).wait()
                            wait_prev_scatter()
                            pltpu.async_copy(h4_hbm.at[gsl1.at[0, 0]],
                                             rows[p ^ 1], sem_r[p ^ 1])
                        else:
                            @pl.when(i < NPAIR - 1)
                            def _():
                                pltpu.make_async_copy(
                                    gs_hbm.at[c, wid, 2 * i + 2], gsl0, si0
                                ).wait()
                                wait_prev_scatter()
                                pltpu.async_copy(h4_hbm.at[gsl0.at[0, 0]],
                                                 rows[p ^ 1], sem_r[p ^ 1])
                        pltpu.make_async_copy(h4_hbm.at[myg.at[0, j]],
                                              rows[p], sem_r[p]).wait()
                        pltpu.async_copy(rows[p], acc.at[myg.at[1, j]],
                                        sem_s[p], add=True)
                    @pl.when(i < NPAIR - 1)
                    def _():
                        pltpu.async_copy(
                            gs_hbm.at[c, wid, 2 * i + gpar + 2],
                            gsl[gpar], sem_i[gpar])
                return carry

            lax.fori_loop(0, NPAIR, pair, 0)
            pltpu.make_async_copy(rows[0], acc.at[gsl1.at[1, 6]],
                                  sem_s[0]).wait()
            pltpu.make_async_copy(rows[1], acc.at[gsl1.at[1, 7]],
                                  sem_s[1]).wait()
            plsc.subcore_barrier()
            a_sl, o_sl = wb_slices(c)
            pltpu.async_copy(a_sl, o_sl, sw)
            if c < C - 1:
                prologue(c + 1)
                pltpu.make_async_copy(a_sl, o_sl, sw).wait()
                zero_issue()
                zero_wait()
                plsc.subcore_barrier()
            else:
                pltpu.make_async_copy(a_sl, o_sl, sw).wait()

    return k


NG2 = 20
NPAIR2 = NG2 // 2
EWP2 = NG2 * GB * K
PAD2 = EWP2 - E // 16


def _sc_segment_sum_fsplit():
    mesh = plsc.VectorSubcoreMesh(core_axis_name="c", subcore_axis_name="s")

    @functools.partial(
        pl.kernel,
        out_type=jax.ShapeDtypeStruct((4, NP, 128), jnp.float32),
        mesh=mesh,
        scratch_types=[
            pltpu.VMEM((ZR, 128), jnp.float32),
            pltpu.VMEM((2, GB, K), jnp.int32),
            pltpu.VMEM((2, GB, K), jnp.int32),
            pltpu.VMEM((K, 128), jnp.float32),
            pltpu.VMEM((K, 128), jnp.float32),
            pltpu.VMEM_SHARED((NP, 128), jnp.float32),
            pltpu.SemaphoreType.DMA,
            pltpu.SemaphoreType.DMA,
            pltpu.SemaphoreType.DMA,
            pltpu.SemaphoreType.DMA,
            pltpu.SemaphoreType.DMA,
            pltpu.SemaphoreType.DMA,
            pltpu.SemaphoreType.DMA,
            pltpu.SemaphoreType.DMA,
        ],
    )
    def k(h4_hbm, gs_hbm, out_hbm, zbuf, gsl0, gsl1, rowsA, rowsB, acc,
          si0, si1, srA, srB, sz, sw, ssA, ssB):
        cid = lax.axis_index("c")
        sid = lax.axis_index("s")
        row0 = sid * STRIPE
        gsl = (gsl0, gsl1)
        sem_i = (si0, si1)
        rows = (rowsA, rowsB)
        sem_r = (srA, srB)
        sem_s = (ssA, ssB)

        def zrow(r, carry):
            for j in range(8):
                zbuf[r, pl.ds(j * 16, 16)] = jnp.zeros((16,), jnp.float32)
            return carry

        lax.fori_loop(0, ZR, zrow, 0)

        def zero_issue():
            for z in range(STRIPE // ZR):
                pltpu.async_copy(zbuf, acc.at[pl.ds(row0 + z * ZR, ZR)], sz)

        def zero_wait():
            for z in range(STRIPE // ZR):
                pltpu.make_async_copy(
                    zbuf, acc.at[pl.ds(row0 + z * ZR, ZR)], sz).wait()

        def prologue(ch):
            pltpu.sync_copy(gs_hbm.at[ch, sid, 0], gsl0)
            pltpu.async_copy(gs_hbm.at[ch, sid, 1], gsl1, si1)
            pltpu.async_copy(h4_hbm.at[gsl0.at[0, 0]], rowsA, srA)

        for cc in range(2):
            ch = cid * 2 + cc
            if cc == 0:
                zero_issue()
                prologue(ch)
                zero_wait()
                plsc.subcore_barrier()

            def pair(i, carry):
                for gpar in (0, 1):
                    myg = gsl[gpar]
                    for j in range(GB):
                        p = j % 2

                        def wait_prev_scatter():
                            pltpu.make_async_copy(
                                rows[p ^ 1], acc.at[myg.at[1, j]],
                                sem_s[p ^ 1]).wait()

                        if j < GB - 1:
                            if j == 0 and gpar == 0:
                                @pl.when(i > 0)
                                def _():
                                    wait_prev_scatter()
                            else:
                                wait_prev_scatter()
                            pltpu.async_copy(h4_hbm.at[myg.at[0, j + 1]],
                                             rows[p ^ 1], sem_r[p ^ 1])
                        elif gpar == 0:
                            pltpu.make_async_copy(
                                gs_hbm.at[ch, sid, 2 * i + 1], gsl1, si1
                            ).wait()
                            wait_prev_scatter()
                            pltpu.async_copy(h4_hbm.at[gsl1.at[0, 0]],
                                             rows[p ^ 1], sem_r[p ^ 1])
                        else:
                            @pl.when(i < NPAIR2 - 1)
                            def _():
                                pltpu.make_async_copy(
                                    gs_hbm.at[ch, sid, 2 * i + 2], gsl0, si0
                                ).wait()
                                wait_prev_scatter()
                                pltpu.async_copy(h4_hbm.at[gsl0.at[0, 0]],
                                                 rows[p ^ 1], sem_r[p ^ 1])
                        pltpu.make_async_copy(h4_hbm.at[myg.at[0, j]],
                                              rows[p], sem_r[p]).wait()
                        pltpu.async_copy(rows[p], acc.at[myg.at[1, j]],
                                         sem_s[p], add=True)

                    @pl.when(i < NPAIR2 - 1)
                    def _():
                        pltpu.async_copy(
                            gs_hbm.at[ch, sid, 2 * i + gpar + 2],
                            gsl[gpar], sem_i[gpar])
                return carry

            lax.fori_loop(0, NPAIR2, pair, 0)
            pltpu.make_async_copy(rows[0], acc.at[gsl1.at[1, 6]],
                                  sem_s[0]).wait()
            pltpu.make_async_copy(rows[1], acc.at[gsl1.at[1, 7]],
                                  sem_s[1]).wait()
            plsc.subcore_barrier()
            a_sl = acc.at[pl.ds(row0, STRIPE)]
            o_sl = out_hbm.at[ch, pl.ds(row0, STRIPE)]
            pltpu.async_copy(a_sl, o_sl, sw)
            if cc == 0:
                prologue(cid * 2 + 1)
                pltpu.make_async_copy(a_sl, o_sl, sw).wait()
                zero_issue()
                zero_wait()
                plsc.subcore_barrier()
            else:
                pltpu.make_async_copy(a_sl, o_sl, sw).wait()

    return k


def _layer_body(nblk, C, outer, two_partials, h_ref, p_ref, w1_ref, w2_ref,
                gm_ref, bm_ref, go_ref, bo_ref, out_ref, z_scr, y_scr,
                st1, st2):
    ph = pl.program_id(0)
    i = pl.program_id(1)
    bn = h_ref.shape[0]

    @pl.when(ph == 0)
    def _():
        h = h_ref[...]
        p = p_ref[...]
        z = jnp.zeros((bn, w1_ref.shape[2]), jnp.float32)
        for c in range(C):
            agg = (p[0, c] + p[1, c]) if two_partials else p[c]
            hc = h[:, c * 128:(c + 1) * 128] + agg
            z = z + jnp.dot(hc, w1_ref[c], preferred_element_type=jnp.float32)
        z_scr[pl.ds(i * bn, bn), :] = z
        ss = jnp.concatenate([jnp.sum(z, 0, keepdims=True),
                              jnp.sum(z * z, 0, keepdims=True)], 0)

        @pl.when(i == 0)
        def _():
            st1[...] = ss

        @pl.when(i > 0)
        def _():
            st1[...] += ss

    @pl.when(ph == 1)
    def _():
        z = z_scr[pl.ds(i * bn, bn), :]
        mean = st1[0:1, :] / N
        var = st1[1:2, :] / N - mean * mean
        inv = lax.rsqrt(var + EPS) * gm_ref[...]
        a = jnp.maximum((z - mean) * inv + bm_ref[...], 0.0)
        y = jnp.dot(a, w2_ref[...], preferred_element_type=jnp.float32)
        if outer:
            z_scr[pl.ds(i * bn, bn), :] = y
            ss = jnp.concatenate([jnp.sum(y, 0, keepdims=True),
                                  jnp.sum(y * y, 0, keepdims=True)], 0)

            @pl.when(i == 0)
            def _():
                st2[...] = ss

            @pl.when(i > 0)
            def _():
                st2[...] += ss
        else:
            out_ref[...] = y

    if outer:
        @pl.when(ph == 2)
        def _():
            y = z_scr[pl.ds(i * bn, bn), :]
            mean = st2[0:1, :] / N
            var = st2[1:2, :] / N - mean * mean
            inv = lax.rsqrt(var + EPS) * go_ref[...]
            out_ref[...] = jnp.maximum((y - mean) * inv + bo_ref[...], 0.0)


def _gin_layer(h, parts, W1c, W2, gm, bm, go, bo, C, outer, bn):
    two_partials = parts.ndim == 4
    nblk = N // bn
    din = C * 128
    dout = W2.shape[1]
    nph = 3 if outer else 2
    last = nblk - 1

    def park0(ph, i):
        return (jnp.where(ph == 0, i, last), 0)

    def park0_4d(ph, i):
        return (0, 0, jnp.where(ph == 0, i, last), 0)

    def park0_3d(ph, i):
        return (0, jnp.where(ph == 0, i, last), 0)

    def outmap(ph, i):
        return (jnp.where(ph == nph - 1, i, 0), 0)

    return pl.pallas_call(
        functools.partial(_layer_body, nblk, C, outer, two_partials),
        grid=(nph, nblk),
        in_specs=[
            pl.BlockSpec((bn, din), park0),
            (pl.BlockSpec((2, C, bn, 128), park0_4d) if two_partials
             else pl.BlockSpec((C, bn, 128), park0_3d)),
            pl.BlockSpec((C, 128, 512), lambda ph, i: (0, 0, 0)),
            pl.BlockSpec((512, dout), lambda ph, i: (0, 0)),
            pl.BlockSpec((1, 512), lambda ph, i: (0, 0)),
            pl.BlockSpec((1, 512), lambda ph, i: (0, 0)),
            pl.BlockSpec((1, 512), lambda ph, i: (0, 0)),
            pl.BlockSpec((1, 512), lambda ph, i: (0, 0)),
        ],
        out_specs=pl.BlockSpec((bn, dout), outmap),
        out_shape=jax.ShapeDtypeStruct((N, dout), jnp.float32),
        scratch_shapes=[
            pltpu.VMEM((N, 512), jnp.float32),
            pltpu.VMEM((8, 512), jnp.float32),
            pltpu.VMEM((2, 512), jnp.float32),
            pltpu.VMEM((2, 512), jnp.float32),
        ],
    )(h, parts, W1c, W2, gm, bm, go, bo)


def kernel(x, edge_index, W1_0, W2_0, gm0, bm0, go0, bo0,
           W1_1, W2_1, gm1, bm1, go1, bo1, W1_2, W2_2, gm2, bm2):
    src = edge_index[0]
    dst = edge_index[1]
    pad_g = jnp.broadcast_to((jnp.arange(PAD, dtype=jnp.int32) * 41) % N,
                             (NW, PAD))
    pad_s = jnp.broadcast_to(N + jnp.arange(PAD, dtype=jnp.int32), (NW, PAD))
    srcp = jnp.concatenate([src.reshape(NW, EW), pad_g], 1)
    dstp = jnp.concatenate([dst.reshape(NW, EW), pad_s], 1)

    def _gs(C):
        g = (srcp[None] * C
             + jnp.arange(C, dtype=jnp.int32)[:, None, None])
        s = jnp.broadcast_to(dstp[None], (C, NW, EWP))
        return jnp.stack([g.reshape(C, NW, NG, GB, K),
                          s.reshape(C, NW, NG, GB, K)], axis=3)

    gs1 = _gs(1)
    pad_g2 = jnp.broadcast_to((jnp.arange(PAD2, dtype=jnp.int32) * 41) % N,
                              (16, PAD2))
    pad_s2 = jnp.broadcast_to(N + (jnp.arange(PAD2, dtype=jnp.int32)
                                   % (NP - N)), (16, PAD2))
    srcp2 = jnp.concatenate([src.reshape(16, E // 16), pad_g2], 1)
    dstp2 = jnp.concatenate([dst.reshape(16, E // 16), pad_s2], 1)
    g2 = srcp2[None] * 4 + jnp.arange(4, dtype=jnp.int32)[:, None, None]
    s2 = jnp.broadcast_to(dstp2[None], (4, 16, EWP2))
    gs4 = jnp.stack([g2.reshape(4, 16, NG2, GB, K),
                     s2.reshape(4, 16, NG2, GB, K)], axis=3)

    seg1 = _sc_segment_sum(1)
    seg4 = _sc_segment_sum_fsplit()
    r2 = lambda v: v.reshape(1, -1)
    BN_ROWS = 2000
    zeros = jnp.zeros((1, 512), jnp.float32)

    p = seg1(x, gs1)
    h = _gin_layer(x, p, W1_0.reshape(1, 128, 512), W2_0, r2(gm0), r2(bm0),
                   r2(go0), r2(bo0), 1, True, BN_ROWS)

    p = seg4(h.reshape(N * 4, 128), gs4)
    h = _gin_layer(h, p, W1_1.reshape(4, 128, 512), W2_1, r2(gm1), r2(bm1),
                   r2(go1), r2(bo1), 4, True, BN_ROWS)

    p = seg4(h.reshape(N * 4, 128), gs4)
    h = _gin_layer(h, p, W1_2.reshape(4, 128, 512), W2_2, r2(gm2), r2(bm2),
                   zeros, zeros, 4, False, BN_ROWS)
    return h
